# baseline (speedup 1.0000x reference)
import functools

import jax
import jax.numpy as jnp
from jax import lax
from jax.experimental import pallas as pl
from jax.experimental.pallas import tpu as pltpu
from jax.experimental.pallas import tpu_sc as plsc

N_ROWS = 1_000_000
N_COLS = 64
TOTAL = N_ROWS * N_COLS
LANES = 16

NUM_WORKERS = 32
PER_WORKER = TOTAL // NUM_WORKERS
CHUNK = 40_000
N_CHUNKS = PER_WORKER // CHUNK


def _make_sc_kernel():
    info = plsc.get_sparse_core_info()
    nc = info.num_cores

    @functools.partial(
        pl.kernel,
        out_type=jax.ShapeDtypeStruct((TOTAL,), jnp.float32),
        mesh=plsc.VectorSubcoreMesh(core_axis_name="c", subcore_axis_name="s"),
        scratch_types=[
            pltpu.VMEM((CHUNK,), jnp.float32),
            pltpu.SemaphoreType.DMA,
        ],
    )
    def add_one_scatter(x_hbm, out_hbm, buf, sem):
        wid = lax.axis_index("s") * nc + lax.axis_index("c")
        base = wid * PER_WORKER

        def chunk_body(i, _):
            off = base + i * CHUNK
            pltpu.async_copy(x_hbm.at[pl.ds(off, CHUNK)], buf, sem).wait()

            def vec_body(j, _):
                sl = pl.ds(j * LANES, LANES)
                buf[sl] = buf[sl] + 1.0
                return 0

            lax.fori_loop(0, CHUNK // LANES, vec_body, 0, unroll=8)

            @pl.when(jnp.logical_and(wid == 0, i == 0))
            def _():
                twos = jnp.full((LANES,), 2.0, dtype=jnp.float32)
                for s in range(4):
                    buf[pl.ds(64 + s * LANES, LANES)] = twos

            pltpu.async_copy(buf, out_hbm.at[pl.ds(off, CHUNK)], sem).wait()
            return 0

        lax.fori_loop(0, N_CHUNKS, chunk_body, 0)

    return add_one_scatter


_sc_add_one = _make_sc_kernel()


@jax.jit
def kernel(x):
    out_flat = _sc_add_one(x.reshape(TOTAL))
    return out_flat.reshape(N_ROWS, N_COLS)


# speedup vs baseline: 1.0699x; 1.0699x over previous
import functools

import jax
import jax.numpy as jnp
from jax import lax
from jax.experimental import pallas as pl
from jax.experimental.pallas import tpu as pltpu
from jax.experimental.pallas import tpu_sc as plsc

N_ROWS = 1_000_000
N_COLS = 64
TOTAL = N_ROWS * N_COLS
LANES = 16

NUM_WORKERS = 32
PER_WORKER = TOTAL // NUM_WORKERS
CHUNK = 40_000
N_CHUNKS = PER_WORKER // CHUNK
NBUF = 3
N_MAIN = N_CHUNKS - 2
assert N_MAIN % NBUF == 0


def _make_sc_kernel():
    info = plsc.get_sparse_core_info()
    nc = info.num_cores

    @functools.partial(
        pl.kernel,
        out_type=jax.ShapeDtypeStruct((TOTAL,), jnp.float32),
        mesh=plsc.VectorSubcoreMesh(core_axis_name="c", subcore_axis_name="s"),
        scratch_types=[
            pltpu.VMEM((CHUNK,), jnp.float32),
            pltpu.VMEM((CHUNK,), jnp.float32),
            pltpu.VMEM((CHUNK,), jnp.float32),
            pltpu.SemaphoreType.DMA,
            pltpu.SemaphoreType.DMA,
            pltpu.SemaphoreType.DMA,
            pltpu.SemaphoreType.DMA,
            pltpu.SemaphoreType.DMA,
            pltpu.SemaphoreType.DMA,
        ],
    )
    def add_one_scatter(x_hbm, out_hbm, b0, b1, b2, l0, l1, l2, s0, s1, s2):
        bufs = (b0, b1, b2)
        lsems = (l0, l1, l2)
        ssems = (s0, s1, s2)
        wid = lax.axis_index("s") * nc + lax.axis_index("c")
        base = wid * PER_WORKER

        def start_load(i, b):
            pltpu.async_copy(
                x_hbm.at[pl.ds(base + i * CHUNK, CHUNK)], bufs[b], lsems[b]
            )

        def start_store(i, b):
            pltpu.async_copy(
                bufs[b], out_hbm.at[pl.ds(base + i * CHUNK, CHUNK)], ssems[b]
            )

        def wait_load(b):
            pltpu.make_async_copy(
                x_hbm.at[pl.ds(0, CHUNK)], bufs[b], lsems[b]
            ).wait()

        def wait_store(b):
            pltpu.make_async_copy(
                bufs[b], out_hbm.at[pl.ds(0, CHUNK)], ssems[b]
            ).wait()

        def compute(i, b):
            def vec_body(j, _):
                sl = pl.ds(j * LANES, LANES)
                bufs[b][sl] = bufs[b][sl] + 1.0
                return 0

            lax.fori_loop(0, CHUNK // LANES, vec_body, 0, unroll=8)

            @pl.when(jnp.logical_and(wid == 0, i == 0))
            def _():
                twos = jnp.full((LANES,), 2.0, dtype=jnp.float32)
                for sft in range(4):
                    bufs[b][pl.ds(64 + sft * LANES, LANES)] = twos

        for b in range(NBUF):
            start_load(b, b)

        def group_body(g, _):
            i0 = g * NBUF
            for b in range(NBUF):
                i = i0 + b
                bn = (b + 2) % NBUF
                @pl.when(i >= 1)
                def _():
                    wait_store(bn)

                start_load(i + 2, bn)
                wait_load(b)
                compute(i, b)
                start_store(i, b)
            return 0

        lax.fori_loop(0, N_MAIN // NBUF, group_body, 0)

        for k in range(2):
            i = N_MAIN + k
            wait_load(i % NBUF)
            compute(i, i % NBUF)
            start_store(i, i % NBUF)
        for b in range(NBUF):
            wait_store(b)

    return add_one_scatter


_sc_add_one = _make_sc_kernel()


@jax.jit
def kernel(x):
    out_flat = _sc_add_one(x.reshape(TOTAL))
    return out_flat.reshape(N_ROWS, N_COLS)
